# baseline (speedup 1.0000x reference)
import jax
import jax.numpy as jnp
from jax import lax
from jax.experimental import pallas as pl
from jax.experimental.pallas import tpu as pltpu
from jax.experimental.pallas import tpu_sc as plsc

_NC = 2
_NS = 16
_NW = _NC * _NS
_L = 16
_KF = 2


def _ln_rows(xb, pb, gb, bb, ob, nrows, d):
    nv = d // _L

    @plsc.parallel_loop(0, nrows, unroll=4)
    def row_body(i):
        e = []
        for k in range(nv):
            e.append(xb[i, pl.ds(k * _L, _L)] + pb[i, pl.ds(k * _L, _L)])
        s = (e[0] + e[1]) + (e[2] + e[3])
        q = (e[0] * e[0] + e[1] * e[1]) + (e[2] * e[2] + e[3] * e[3])
        lanes = lax.iota(jnp.int32, _L)
        for sh in (8, 4, 2, 1):
            perm = lanes ^ sh
            s = s + s.at[perm].get(mode="promise_in_bounds")
            q = q + q.at[perm].get(mode="promise_in_bounds")
        mean = s * (1.0 / d)
        var = q * (1.0 / d) - mean * mean
        t = var + 1e-5
        bits = lax.bitcast_convert_type(t, jnp.int32)
        y = lax.bitcast_convert_type(
            jnp.int32(0x5F3759DF) - lax.shift_right_logical(bits, 1),
            jnp.float32,
        )
        y = y * (1.5 - 0.5 * t * y * y)
        y = y * (1.5 - 0.5 * t * y * y)
        y = y * (1.5 - 0.5 * t * y * y)
        for k in range(nv):
            a = y * gb[pl.ds(k * _L, _L)]
            o = (e[k] - mean) * a + bb[pl.ds(k * _L, _L)]
            ob[i, pl.ds(k * _L, _L)] = o


def _sc_body(x_hbm, pe_hbm, g_hbm, b_hbm, o_hbm,
             peb, gb, bb, xb0, xb1, ob0, ob1,
             sem_in0, sem_in1, sem_out0, sem_out1):
    b, nf, s, d = x_hbm.shape
    ngrp = (b * nf) // _KF
    gper = nf // _KF
    stripe = s // _NW

    wid = lax.axis_index("c") * _NS + lax.axis_index("s")
    r0 = wid * stripe

    pltpu.sync_copy(pe_hbm.at[pl.ds(r0, stripe)], peb)
    pltpu.sync_copy(g_hbm, gb)
    pltpu.sync_copy(b_hbm, bb)

    xbufs = (xb0, xb1)
    obufs = (ob0, ob1)
    isems = (sem_in0, sem_in1)
    osems = (sem_out0, sem_out1)

    def in_copy(grp, p):
        bi = grp // gper
        fi = lax.rem(grp, gper) * _KF
        return pltpu.make_async_copy(
            x_hbm.at[bi, pl.ds(fi, _KF), pl.ds(r0, stripe)], xbufs[p],
            isems[p])

    def out_copy(grp, p):
        bi = grp // gper
        fi = lax.rem(grp, gper) * _KF
        return pltpu.make_async_copy(
            obufs[p], o_hbm.at[bi, pl.ds(fi, _KF), pl.ds(r0, stripe)],
            osems[p])

    in_copy(0, 0).start()
    in_copy(1, 1).start()

    def grp_pair_body(j, carry):
        del carry
        for p in range(2):
            i = 2 * j + p
            in_copy(i, p).wait()

            @pl.when(j > 0)
            def _():
                out_copy(i - 2, p).wait()

            for kf in range(_KF):
                _ln_rows(xbufs[p].at[kf], peb, gb, bb, obufs[p].at[kf],
                         stripe, d)
            out_copy(i, p).start()

            @pl.when(i + 2 < ngrp)
            def _():
                in_copy(i + 2, p).start()

        return 0

    lax.fori_loop(0, ngrp // 2, grp_pair_body, 0)

    out_copy(ngrp - 2, 0).wait()
    out_copy(ngrp - 1, 1).wait()


def kernel(x, pos_embed, gamma, beta, batch_size):
    del batch_size
    b, nf, s, d = x.shape
    stripe = s // _NW

    mesh = plsc.VectorSubcoreMesh(core_axis_name="c", subcore_axis_name="s")
    f = pl.kernel(
        _sc_body,
        out_type=jax.ShapeDtypeStruct((b, nf, s, d), jnp.float32),
        mesh=mesh,
        scratch_types=[
            pltpu.VMEM((stripe, d), jnp.float32),
            pltpu.VMEM((d,), jnp.float32),
            pltpu.VMEM((d,), jnp.float32),
            pltpu.VMEM((_KF, stripe, d), jnp.float32),
            pltpu.VMEM((_KF, stripe, d), jnp.float32),
            pltpu.VMEM((_KF, stripe, d), jnp.float32),
            pltpu.VMEM((_KF, stripe, d), jnp.float32),
            pltpu.SemaphoreType.DMA,
            pltpu.SemaphoreType.DMA,
            pltpu.SemaphoreType.DMA,
            pltpu.SemaphoreType.DMA,
        ],
    )
    return f(x, pos_embed, gamma, beta)


# speedup vs baseline: 2.0266x; 2.0266x over previous
import jax
import jax.numpy as jnp
from jax.experimental import pallas as pl
from jax.experimental.pallas import tpu as pltpu


def _ln_body(x_ref, pe_ref, g_ref, b_ref, o_ref):
    e = x_ref[...] + pe_ref[...]
    m = jnp.mean(e, axis=-1, keepdims=True)
    c = e - m
    v = jnp.mean(c * c, axis=-1, keepdims=True)
    inv = jax.lax.rsqrt(v + 1e-5)
    o_ref[...] = c * inv * g_ref[...] + b_ref[...]


def kernel(x, pos_embed, gamma, beta, batch_size):
    del batch_size
    b, nf, s, d = x.shape
    rows = b * nf
    xr = x.reshape(rows, s, d)
    g = gamma.reshape(1, 1, d)
    bt = beta.reshape(1, 1, d)

    G = 4
    grid = (rows // G,)

    out = pl.pallas_call(
        _ln_body,
        grid=grid,
        in_specs=[
            pl.BlockSpec((G, s, d), lambda i: (i, 0, 0)),
            pl.BlockSpec((s, d), lambda i: (0, 0)),
            pl.BlockSpec((1, 1, d), lambda i: (0, 0, 0)),
            pl.BlockSpec((1, 1, d), lambda i: (0, 0, 0)),
        ],
        out_specs=pl.BlockSpec((G, s, d), lambda i: (i, 0, 0)),
        out_shape=jax.ShapeDtypeStruct((rows, s, d), x.dtype),
        compiler_params=pltpu.CompilerParams(
            dimension_semantics=("parallel",),
        ),
    )(xr, pos_embed, g, bt)
    return out.reshape(b, nf, s, d)


# speedup vs baseline: 2.1292x; 1.0506x over previous
import jax
import jax.numpy as jnp
from jax.experimental import pallas as pl
from jax.experimental.pallas import tpu as pltpu


def _ln_body(x_ref, pe_ref, g_ref, b_ref, o_ref):
    e = x_ref[...] + pe_ref[...]
    m = jnp.mean(e, axis=-1, keepdims=True)
    c = e - m
    v = jnp.mean(c * c, axis=-1, keepdims=True)
    inv = jax.lax.rsqrt(v + 1e-5)
    o_ref[...] = c * inv * g_ref[...] + b_ref[...]


def kernel(x, pos_embed, gamma, beta, batch_size):
    del batch_size
    b, nf, s, d = x.shape
    rows = b * nf
    xr = x.reshape(rows, s, d)
    g = gamma.reshape(1, 1, d)
    bt = beta.reshape(1, 1, d)

    G = 8
    grid = (rows // G,)

    out = pl.pallas_call(
        _ln_body,
        grid=grid,
        in_specs=[
            pl.BlockSpec((G, s, d), lambda i: (i, 0, 0)),
            pl.BlockSpec((s, d), lambda i: (0, 0)),
            pl.BlockSpec((1, 1, d), lambda i: (0, 0, 0)),
            pl.BlockSpec((1, 1, d), lambda i: (0, 0, 0)),
        ],
        out_specs=pl.BlockSpec((G, s, d), lambda i: (i, 0, 0)),
        out_shape=jax.ShapeDtypeStruct((rows, s, d), x.dtype),
        compiler_params=pltpu.CompilerParams(
            dimension_semantics=("parallel",),
        ),
    )(xr, pos_embed, g, bt)
    return out.reshape(b, nf, s, d)
